# Initial kernel scaffold; baseline (speedup 1.0000x reference)
import jax
import jax.numpy as jnp
from jax.experimental import pallas as pl


def kernel(x, edge_index_all, rel_all, rel_emb, W1):
    raise NotImplementedError("write your pallas kernel here")


# speedup vs baseline: 3.8678x; 3.8678x over previous
import functools

import jax
import jax.numpy as jnp
from jax import lax
from jax.experimental import pallas as pl
from jax.experimental.pallas import tpu as pltpu
from jax.experimental.pallas import tpu_sc as plsc

N_NODES = 10000
N_EDGES = 160000
N_RELS = 1000
D_FEAT = 300
R_HID = 100

NC, NS, L = 2, 16, 16
EPT = N_EDGES // NS
CHUNK = 80
NCHUNK = EPT // CHUNK
NPAD = 10240
ROWS_T = NPAD // NS


def _prep_body(x_ref, rel_ref, w1_ref, ha_ref, hb_ref, hc_ref, rtab_ref,
               cpr_ref, bpr_ref):
    x = x_ref[...]
    h = jnp.where(x >= 0, x, 0.01 * x)
    zpad = jnp.zeros((h.shape[0], 28), jnp.float32)
    ha_ref[...] = jnp.concatenate([h[:, 0:100], zpad], axis=1)
    hb_ref[...] = jnp.concatenate([h[:, 100:200], zpad], axis=1)
    hc_ref[...] = jnp.concatenate([h[:, 200:300], zpad], axis=1)
    w1 = w1_ref[...]
    wc = w1[:, 400:700]
    cvec = jnp.dot(h, wc.T, preferred_element_type=jnp.float32)
    cpr_ref[...] = cvec - jnp.max(cvec)
    rel = rel_ref[...]
    r = jnp.where(rel >= 0, rel, 0.01 * rel)
    rtab_ref[...] = jnp.concatenate(
        [r, jnp.zeros((r.shape[0], 28), jnp.float32)], axis=1)
    wb = w1[:, 300:400]
    bvec = jnp.dot(r, wb.T, preferred_element_type=jnp.float32)
    bpr_ref[...] = bvec - jnp.max(bvec)


_prep = pl.pallas_call(
    _prep_body,
    out_shape=[
        jax.ShapeDtypeStruct((N_NODES, 128), jnp.float32),
        jax.ShapeDtypeStruct((N_NODES, 128), jnp.float32),
        jax.ShapeDtypeStruct((N_NODES, 128), jnp.float32),
        jax.ShapeDtypeStruct((N_RELS, 128), jnp.float32),
        jax.ShapeDtypeStruct((N_NODES, 1), jnp.float32),
        jax.ShapeDtypeStruct((N_RELS, 1), jnp.float32),
    ],
)


def _head_body(ha_ref, hb_ref, hc_ref, den_ref, out_ref):
    den = den_ref[...]
    s = den / (den + 1e-16)
    out_ref[:, 0:100] = ha_ref[:, 0:100] * s
    out_ref[:, 100:200] = hb_ref[:, 0:100] * s
    out_ref[:, 200:300] = hc_ref[:, 0:100] * s


_head = pl.pallas_call(
    _head_body,
    out_shape=jax.ShapeDtypeStruct((N_NODES, D_FEAT), jnp.float32),
)


_mesh = plsc.VectorSubcoreMesh(
    core_axis_name="c", subcore_axis_name="s", num_cores=NC, num_subcores=NS
)


def _make_sc_main(body, interpret=False):
    return pl.kernel(
        body,
        mesh=_mesh,
        interpret=interpret,
        compiler_params=pltpu.CompilerParams(
            needs_layout_passes=False, use_tc_tiling_on_sc=False
        ),
        out_type=[
        jax.ShapeDtypeStruct((NPAD, 128), jnp.float32),
        jax.ShapeDtypeStruct((NPAD, 128), jnp.float32),
        jax.ShapeDtypeStruct((NPAD, 128), jnp.float32),
        jax.ShapeDtypeStruct((NPAD, 128), jnp.float32),
        jax.ShapeDtypeStruct((NPAD,), jnp.float32),
    ],
    scratch_types=[
        pltpu.VMEM_SHARED((NPAD, 128), jnp.float32),
        pltpu.VMEM_SHARED((NPAD,), jnp.float32),
        pltpu.VMEM((N_NODES,), jnp.float32),
        pltpu.VMEM((N_RELS,), jnp.float32),
        pltpu.VMEM((NPAD,), jnp.float32),
        pltpu.VMEM((CHUNK,), jnp.int32),
        pltpu.VMEM((CHUNK,), jnp.int32),
        pltpu.VMEM((CHUNK,), jnp.int32),
        pltpu.VMEM((CHUNK,), jnp.float32),
        pltpu.VMEM((CHUNK, 128), jnp.float32),
        pltpu.SemaphoreType.DMA,
        ],
    )


def _sc_body(iarr, jarr, rarr, cpr, bpr, ha, hb, hc, rtab, zrows, zden,
             outa0, outb0, outa1, outb1, dout,
             acc, den_sp, cpr_v, bpr_v, den_v,
             ibuf, jbuf, rbuf, ebuf, rows, sem):
    cid = lax.axis_index("c")
    sid = lax.axis_index("s")

    def _zero_acc():
        pltpu.sync_copy(zrows, acc.at[pl.ds(sid * ROWS_T, ROWS_T)])

    def _load_idx(g):
        base = sid * EPT + g * CHUNK
        pltpu.sync_copy(iarr.at[pl.ds(base, CHUNK)], ibuf)
        pltpu.sync_copy(jarr.at[pl.ds(base, CHUNK)], jbuf)
        pltpu.sync_copy(rarr.at[pl.ds(base, CHUNK)], rbuf)

    pltpu.sync_copy(cpr, cpr_v)
    pltpu.sync_copy(bpr, bpr_v)
    _zero_acc()
    pltpu.sync_copy(zden.at[pl.ds(sid * (NPAD // NS), NPAD // NS)],
                    den_sp.at[pl.ds(sid * (NPAD // NS), NPAD // NS)])
    plsc.subcore_barrier()

    def _p1(g, carry):
        _load_idx(g)
        for k in range(CHUNK // L):
            j16 = jbuf[pl.ds(k * L, L)]
            r16 = rbuf[pl.ds(k * L, L)]
            cv = plsc.load_gather(cpr_v, [j16])
            bv = plsc.load_gather(bpr_v, [r16])
            ebuf[pl.ds(k * L, L)] = jnp.exp(bv + cv)
        pltpu.sync_copy(ebuf, den_sp.at[ibuf], add=True)
        return carry

    lax.fori_loop(0, NCHUNK, _p1, 0)
    plsc.subcore_barrier()

    pltpu.sync_copy(den_sp, den_v)

    @pl.when(cid == 0)
    def _():
        pltpu.sync_copy(den_sp.at[pl.ds(sid * (NPAD // NS), NPAD // NS)],
                        dout.at[pl.ds(sid * (NPAD // NS), NPAD // NS)])

    def _subpass(tab0, tab1, use_rel_idx):
        def _body(g, carry):
            _load_idx(g)
            for k in range(CHUNK // L):
                sl = pl.ds(k * L, L)
                i16 = ibuf[sl]
                j16 = jbuf[sl]
                r16 = rbuf[sl]
                cv = plsc.load_gather(cpr_v, [j16])
                bv = plsc.load_gather(bpr_v, [r16])
                dv = plsc.load_gather(den_v, [i16])
                ebuf[sl] = jnp.exp(bv + cv) / (dv + 1e-16)

            @pl.when(cid == 0)
            def _():
                pltpu.async_copy(tab0.at[jbuf], rows, sem).wait()

            @pl.when(cid == 1)
            def _():
                idx = rbuf if use_rel_idx else jbuf
                pltpu.async_copy(tab1.at[idx], rows, sem).wait()

            def _scale(e, c2):
                al = plsc.load_gather(ebuf, [jnp.full((L,), e, jnp.int32)])
                for off in (0, 16, 32, 48, 64, 80, 96):
                    rows[e, pl.ds(off, L)] = rows[e, pl.ds(off, L)] * al
                return c2

            lax.fori_loop(0, CHUNK, _scale, 0)
            pltpu.sync_copy(rows, acc.at[ibuf], add=True)
            return carry

        lax.fori_loop(0, NCHUNK, _body, 0)
        plsc.subcore_barrier()

    def _dump_acc(out0, out1):
        sl = pl.ds(sid * ROWS_T, ROWS_T)

        @pl.when(cid == 0)
        def _():
            pltpu.sync_copy(acc.at[sl], out0.at[sl])

        @pl.when(cid == 1)
        def _():
            pltpu.sync_copy(acc.at[sl], out1.at[sl])

    _subpass(ha, hc, use_rel_idx=False)
    _dump_acc(outa0, outa1)
    _zero_acc()
    plsc.subcore_barrier()

    _subpass(hb, rtab, use_rel_idx=True)
    _dump_acc(outb0, outb1)


_sc_main = _make_sc_main(_sc_body)


def kernel(x, edge_index_all, rel_all, rel_emb, W1):
    i_arr = edge_index_all[0].astype(jnp.int32)
    j_arr = edge_index_all[1].astype(jnp.int32)
    r_arr = rel_all.astype(jnp.int32)

    ha, hb, hc, rtab, cpr, bpr = _prep(x, rel_emb, W1)
    zrows = jnp.zeros((ROWS_T, 128), jnp.float32)
    zden = jnp.zeros((NPAD,), jnp.float32)

    outa0, outb0, outa1, outb1, dout = _sc_main(
        i_arr, j_arr, r_arr,
        cpr.reshape(-1), bpr.reshape(-1),
        ha, hb, hc, rtab, zrows, zden,
    )

    head = _head(ha, hb, hc, dout[:N_NODES].reshape(N_NODES, 1))
    return jnp.concatenate(
        [head, outb1[:N_NODES, :100], outa0[:N_NODES, :100],
         outb0[:N_NODES, :100], outa1[:N_NODES, :100]],
        axis=1,
    )


# speedup vs baseline: 4.7582x; 1.2302x over previous
import functools

import jax
import jax.numpy as jnp
from jax import lax
from jax.experimental import pallas as pl
from jax.experimental.pallas import tpu as pltpu
from jax.experimental.pallas import tpu_sc as plsc

N_NODES = 10000
N_EDGES = 160000
N_RELS = 1000
D_FEAT = 300
R_HID = 100

NC, NS, L = 2, 16, 16
EPT = N_EDGES // NS
CHUNK = 80
NCHUNK = EPT // CHUNK
NPAD = 10240
ROWS_T = NPAD // NS


def _prep_body(x_ref, rel_ref, w1_ref, ha_ref, hb_ref, hc_ref, rtab_ref,
               cpr_ref, bpr_ref):
    x = x_ref[...]
    h = jnp.where(x >= 0, x, 0.01 * x)
    zpad = jnp.zeros((h.shape[0], 28), jnp.float32)
    ha_ref[...] = jnp.concatenate([h[:, 0:100], zpad], axis=1)
    hb_ref[...] = jnp.concatenate([h[:, 100:200], zpad], axis=1)
    hc_ref[...] = jnp.concatenate([h[:, 200:300], zpad], axis=1)
    w1 = w1_ref[...]
    wc = w1[:, 400:700]
    cvec = jnp.dot(h, wc.T, preferred_element_type=jnp.float32)
    cpr_ref[...] = cvec - jnp.max(cvec)
    rel = rel_ref[...]
    r = jnp.where(rel >= 0, rel, 0.01 * rel)
    rtab_ref[...] = jnp.concatenate(
        [r, jnp.zeros((r.shape[0], 28), jnp.float32)], axis=1)
    wb = w1[:, 300:400]
    bvec = jnp.dot(r, wb.T, preferred_element_type=jnp.float32)
    bpr_ref[...] = bvec - jnp.max(bvec)


_prep = pl.pallas_call(
    _prep_body,
    out_shape=[
        jax.ShapeDtypeStruct((N_NODES, 128), jnp.float32),
        jax.ShapeDtypeStruct((N_NODES, 128), jnp.float32),
        jax.ShapeDtypeStruct((N_NODES, 128), jnp.float32),
        jax.ShapeDtypeStruct((N_RELS, 128), jnp.float32),
        jax.ShapeDtypeStruct((N_NODES, 1), jnp.float32),
        jax.ShapeDtypeStruct((N_RELS, 1), jnp.float32),
    ],
)


def _head_body(ha_ref, hb_ref, hc_ref, den_ref, out_ref):
    den = den_ref[...]
    s = den / (den + 1e-16)
    out_ref[:, 0:100] = ha_ref[:, 0:100] * s
    out_ref[:, 100:200] = hb_ref[:, 0:100] * s
    out_ref[:, 200:300] = hc_ref[:, 0:100] * s


_head = pl.pallas_call(
    _head_body,
    out_shape=jax.ShapeDtypeStruct((N_NODES, D_FEAT), jnp.float32),
)


_mesh = plsc.VectorSubcoreMesh(
    core_axis_name="c", subcore_axis_name="s", num_cores=NC, num_subcores=NS
)


def _make_sc_main(body, interpret=False):
    return pl.kernel(
        body,
        mesh=_mesh,
        interpret=interpret,
        compiler_params=pltpu.CompilerParams(
            needs_layout_passes=False, use_tc_tiling_on_sc=False
        ),
        out_type=[
        jax.ShapeDtypeStruct((NPAD, 128), jnp.float32),
        jax.ShapeDtypeStruct((NPAD, 128), jnp.float32),
        jax.ShapeDtypeStruct((NPAD, 128), jnp.float32),
        jax.ShapeDtypeStruct((NPAD, 128), jnp.float32),
        jax.ShapeDtypeStruct((NPAD,), jnp.float32),
    ],
    scratch_types=[
        pltpu.VMEM_SHARED((NPAD, 128), jnp.float32),
        pltpu.VMEM_SHARED((NPAD,), jnp.float32),
        pltpu.VMEM((N_NODES,), jnp.float32),
        pltpu.VMEM((N_RELS,), jnp.float32),
        pltpu.VMEM((NPAD,), jnp.float32),
        pltpu.VMEM((CHUNK,), jnp.int32),
        pltpu.VMEM((CHUNK,), jnp.int32),
        pltpu.VMEM((CHUNK,), jnp.int32),
        pltpu.VMEM((CHUNK,), jnp.float32),
        pltpu.VMEM((CHUNK, 128), jnp.float32),
        pltpu.SemaphoreType.DMA,
        pltpu.VMEM((CHUNK,), jnp.int32),
        pltpu.VMEM((CHUNK,), jnp.int32),
        pltpu.VMEM((CHUNK,), jnp.int32),
        pltpu.VMEM((CHUNK,), jnp.float32),
        pltpu.VMEM((CHUNK, 128), jnp.float32),
        pltpu.SemaphoreType.DMA,
        ],
    )


def _sc_body(iarr, jarr, rarr, cpr, bpr, ha, hb, hc, rtab, zrows, zden,
             outa0, outb0, outa1, outb1, dout,
             acc, den_sp, cpr_v, bpr_v, den_v,
             ibuf, jbuf, rbuf, ebuf, rows, sem,
             ibufB, jbufB, rbufB, ebufB, rowsB, semB):
    cid = lax.axis_index("c")
    sid = lax.axis_index("s")

    def _zero_acc():
        pltpu.sync_copy(zrows, acc.at[pl.ds(sid * ROWS_T, ROWS_T)])

    def _load_idx(g):
        base = sid * EPT + g * CHUNK
        pltpu.sync_copy(iarr.at[pl.ds(base, CHUNK)], ibuf)
        pltpu.sync_copy(jarr.at[pl.ds(base, CHUNK)], jbuf)
        pltpu.sync_copy(rarr.at[pl.ds(base, CHUNK)], rbuf)

    pltpu.sync_copy(cpr, cpr_v)
    pltpu.sync_copy(bpr, bpr_v)
    _zero_acc()
    pltpu.sync_copy(zden.at[pl.ds(sid * (NPAD // NS), NPAD // NS)],
                    den_sp.at[pl.ds(sid * (NPAD // NS), NPAD // NS)])
    plsc.subcore_barrier()

    def _p1(g, carry):
        _load_idx(g)
        for k in range(CHUNK // L):
            j16 = jbuf[pl.ds(k * L, L)]
            r16 = rbuf[pl.ds(k * L, L)]
            cv = plsc.load_gather(cpr_v, [j16])
            bv = plsc.load_gather(bpr_v, [r16])
            ebuf[pl.ds(k * L, L)] = jnp.exp(bv + cv)
        pltpu.sync_copy(ebuf, den_sp.at[ibuf], add=True)
        return carry

    lax.fori_loop(0, NCHUNK, _p1, 0)
    plsc.subcore_barrier()

    pltpu.sync_copy(den_sp, den_v)

    @pl.when(cid == 0)
    def _():
        pltpu.sync_copy(den_sp.at[pl.ds(sid * (NPAD // NS), NPAD // NS)],
                        dout.at[pl.ds(sid * (NPAD // NS), NPAD // NS)])

    def _subpass(tab0, tab1, use_rel_idx):
        bA = (ibuf, jbuf, rbuf, ebuf, rows, sem)
        bB = (ibufB, jbufB, rbufB, ebufB, rowsB, semB)

        def _issue(g, bufset):
            ib, jb, rb, eb, rw, sm = bufset
            base = sid * EPT + g * CHUNK
            pltpu.sync_copy(iarr.at[pl.ds(base, CHUNK)], ib)
            pltpu.sync_copy(jarr.at[pl.ds(base, CHUNK)], jb)
            pltpu.sync_copy(rarr.at[pl.ds(base, CHUNK)], rb)
            for k in range(CHUNK // L):
                sl = pl.ds(k * L, L)
                cv = plsc.load_gather(cpr_v, [jb[sl]])
                bv = plsc.load_gather(bpr_v, [rb[sl]])
                dv = plsc.load_gather(den_v, [ib[sl]])
                eb[sl] = jnp.exp(bv + cv) / (dv + 1e-16)

            @pl.when(cid == 0)
            def _():
                pltpu.async_copy(tab0.at[jb], rw, sm)

            @pl.when(cid == 1)
            def _():
                idx = rb if use_rel_idx else jb
                pltpu.async_copy(tab1.at[idx], rw, sm)

        def _finish(bufset):
            ib, jb, rb, eb, rw, sm = bufset

            @pl.when(cid == 0)
            def _():
                pltpu.make_async_copy(tab0.at[jb], rw, sm).wait()

            @pl.when(cid == 1)
            def _():
                idx = rb if use_rel_idx else jb
                pltpu.make_async_copy(tab1.at[idx], rw, sm).wait()

            def _scale(e, c2):
                al = plsc.load_gather(eb, [jnp.full((L,), e, jnp.int32)])
                for off in (0, 16, 32, 48, 64, 80, 96):
                    rw[e, pl.ds(off, L)] = rw[e, pl.ds(off, L)] * al
                return c2

            lax.fori_loop(0, CHUNK, _scale, 0)
            pltpu.sync_copy(rw, acc.at[ib], add=True)

        _issue(0, bA)

        def _pair(h, carry):
            g = 2 * h
            _issue(g + 1, bB)
            _finish(bA)
            _issue(g + 2, bA)
            _finish(bB)
            return carry

        lax.fori_loop(0, (NCHUNK - 1) // 2, _pair, 0)
        _finish(bA)
        plsc.subcore_barrier()

    def _dump_acc(out0, out1):
        sl = pl.ds(sid * ROWS_T, ROWS_T)

        @pl.when(cid == 0)
        def _():
            pltpu.sync_copy(acc.at[sl], out0.at[sl])

        @pl.when(cid == 1)
        def _():
            pltpu.sync_copy(acc.at[sl], out1.at[sl])

    _subpass(ha, hc, use_rel_idx=False)
    _dump_acc(outa0, outa1)
    _zero_acc()
    plsc.subcore_barrier()

    _subpass(hb, rtab, use_rel_idx=True)
    _dump_acc(outb0, outb1)


_sc_main = _make_sc_main(_sc_body)


def kernel(x, edge_index_all, rel_all, rel_emb, W1):
    i_arr = edge_index_all[0].astype(jnp.int32)
    j_arr = edge_index_all[1].astype(jnp.int32)
    r_arr = rel_all.astype(jnp.int32)

    ha, hb, hc, rtab, cpr, bpr = _prep(x, rel_emb, W1)
    zrows = jnp.zeros((ROWS_T, 128), jnp.float32)
    zden = jnp.zeros((NPAD,), jnp.float32)

    outa0, outb0, outa1, outb1, dout = _sc_main(
        i_arr, j_arr, r_arr,
        cpr.reshape(-1), bpr.reshape(-1),
        ha, hb, hc, rtab, zrows, zden,
    )

    head = _head(ha, hb, hc, dout[:N_NODES].reshape(N_NODES, 1))
    return jnp.concatenate(
        [head, outb1[:N_NODES, :100], outa0[:N_NODES, :100],
         outb0[:N_NODES, :100], outa1[:N_NODES, :100]],
        axis=1,
    )


# speedup vs baseline: 7.3692x; 1.5487x over previous
import functools

import jax
import jax.numpy as jnp
from jax import lax
from jax.experimental import pallas as pl
from jax.experimental.pallas import tpu as pltpu
from jax.experimental.pallas import tpu_sc as plsc

N_NODES = 10000
N_EDGES = 160000
N_RELS = 1000
D_FEAT = 300
R_HID = 100

NC, NS, L = 2, 16, 16
EPT = N_EDGES // NS
CHUNK = 80
NCHUNK = EPT // CHUNK
NPAD = 10240
ROWS_T = NPAD // NS


def _prep_body(x_ref, rel_ref, w1_ref, ha_ref, hb_ref, hc_ref, rtab_ref,
               cpr_ref, bpr_ref):
    x = x_ref[...]
    h = jnp.where(x >= 0, x, 0.01 * x)
    zpad = jnp.zeros((h.shape[0], 28), jnp.float32)
    ha_ref[...] = jnp.concatenate([h[:, 0:100], zpad], axis=1)
    hb_ref[...] = jnp.concatenate([h[:, 100:200], zpad], axis=1)
    hc_ref[...] = jnp.concatenate([h[:, 200:300], zpad], axis=1)
    w1 = w1_ref[...]
    wc = w1[:, 400:700]
    cvec = jnp.dot(h, wc.T, preferred_element_type=jnp.float32)
    cpr_ref[...] = cvec - jnp.max(cvec)
    rel = rel_ref[...]
    r = jnp.where(rel >= 0, rel, 0.01 * rel)
    rtab_ref[...] = jnp.concatenate(
        [r, jnp.zeros((r.shape[0], 28), jnp.float32)], axis=1)
    wb = w1[:, 300:400]
    bvec = jnp.dot(r, wb.T, preferred_element_type=jnp.float32)
    bpr_ref[...] = bvec - jnp.max(bvec)


_prep = pl.pallas_call(
    _prep_body,
    out_shape=[
        jax.ShapeDtypeStruct((N_NODES, 128), jnp.float32),
        jax.ShapeDtypeStruct((N_NODES, 128), jnp.float32),
        jax.ShapeDtypeStruct((N_NODES, 128), jnp.float32),
        jax.ShapeDtypeStruct((N_RELS, 128), jnp.float32),
        jax.ShapeDtypeStruct((N_NODES, 1), jnp.float32),
        jax.ShapeDtypeStruct((N_RELS, 1), jnp.float32),
    ],
)


def _head_body(ha_ref, hb_ref, hc_ref, den_ref, out_ref):
    den = den_ref[...]
    s = den / (den + 1e-16)
    out_ref[:, 0:100] = ha_ref[:, 0:100] * s
    out_ref[:, 100:200] = hb_ref[:, 0:100] * s
    out_ref[:, 200:300] = hc_ref[:, 0:100] * s


_head = pl.pallas_call(
    _head_body,
    out_shape=jax.ShapeDtypeStruct((N_NODES, D_FEAT), jnp.float32),
)


_mesh = plsc.VectorSubcoreMesh(
    core_axis_name="c", subcore_axis_name="s", num_cores=NC, num_subcores=NS
)


def _make_sc_main(body, interpret=False):
    return pl.kernel(
        body,
        mesh=_mesh,
        interpret=interpret,
        compiler_params=pltpu.CompilerParams(
            needs_layout_passes=False, use_tc_tiling_on_sc=False
        ),
        out_type=[
        jax.ShapeDtypeStruct((NPAD, 128), jnp.float32),
        jax.ShapeDtypeStruct((NPAD, 128), jnp.float32),
        jax.ShapeDtypeStruct((NPAD, 128), jnp.float32),
        jax.ShapeDtypeStruct((NPAD, 128), jnp.float32),
        jax.ShapeDtypeStruct((NPAD,), jnp.float32),
    ],
    scratch_types=[
        pltpu.VMEM_SHARED((NPAD, 128), jnp.float32),
        pltpu.VMEM_SHARED((NPAD,), jnp.float32),
        pltpu.VMEM((N_NODES,), jnp.float32),
        pltpu.VMEM((N_RELS,), jnp.float32),
        pltpu.VMEM((NPAD,), jnp.float32),
        pltpu.VMEM((3, CHUNK), jnp.int32),
        pltpu.VMEM((CHUNK,), jnp.float32),
        pltpu.VMEM((CHUNK, 128), jnp.float32),
        pltpu.SemaphoreType.DMA,
        pltpu.VMEM((3, CHUNK), jnp.int32),
        pltpu.VMEM((CHUNK,), jnp.float32),
        pltpu.VMEM((CHUNK, 128), jnp.float32),
        pltpu.SemaphoreType.DMA,
        ],
    )


def _sc_body(packed, cpr, bpr, ha, hb, hc, rtab, zrows, zden,
             outa0, outb0, outa1, outb1, dout,
             acc, den_sp, cpr_v, bpr_v, den_v,
             buf3, ebuf, rows, sem,
             buf3B, ebufB, rowsB, semB):
    cid = lax.axis_index("c")
    sid = lax.axis_index("s")

    bA = (buf3, ebuf, rows, sem)
    bB = (buf3B, ebufB, rowsB, semB)

    def _zero_acc():
        pltpu.sync_copy(zrows, acc.at[pl.ds(sid * ROWS_T, ROWS_T)])

    def _load_idx(g, b3):
        pltpu.sync_copy(packed.at[sid * NCHUNK + g], b3)

    pltpu.sync_copy(cpr, cpr_v)
    pltpu.sync_copy(bpr, bpr_v)
    _zero_acc()
    pltpu.sync_copy(zden.at[pl.ds(sid * (NPAD // NS), NPAD // NS)],
                    den_sp.at[pl.ds(sid * (NPAD // NS), NPAD // NS)])
    plsc.subcore_barrier()

    def _p1_issue(g, bufset):
        b3, eb, _, _ = bufset
        _load_idx(g, b3)
        for k in range(CHUNK // L):
            sl = pl.ds(k * L, L)
            cv = plsc.load_gather(cpr_v, [b3[1, sl]])
            bv = plsc.load_gather(bpr_v, [b3[2, sl]])
            eb[sl] = jnp.exp(bv + cv)

    def _p1_start(bufset):
        b3, eb, _, sm = bufset
        pltpu.async_copy(eb, den_sp.at[b3.at[0]], sm, add=True)

    def _p1_wait(bufset):
        b3, eb, _, sm = bufset
        pltpu.make_async_copy(eb, den_sp.at[b3.at[0]], sm).wait()

    _p1_issue(0, bA)

    def _p1_pair(h, carry):
        g = 2 * h
        _p1_start(bA)
        _p1_issue(g + 1, bB)
        _p1_wait(bA)
        _p1_start(bB)
        _p1_issue(g + 2, bA)
        _p1_wait(bB)
        return carry

    lax.fori_loop(0, (NCHUNK - 1) // 2, _p1_pair, 0)
    _p1_start(bA)
    _p1_wait(bA)
    plsc.subcore_barrier()

    pltpu.sync_copy(den_sp, den_v)

    @pl.when(cid == 0)
    def _():
        pltpu.sync_copy(den_sp.at[pl.ds(sid * (NPAD // NS), NPAD // NS)],
                        dout.at[pl.ds(sid * (NPAD // NS), NPAD // NS)])

    def _subpass(tab0, tab1, use_rel_idx):
        def _issue(g, bufset):
            b3, eb, rw, sm = bufset
            _load_idx(g, b3)
            for k in range(CHUNK // L):
                sl = pl.ds(k * L, L)
                cv = plsc.load_gather(cpr_v, [b3[1, sl]])
                bv = plsc.load_gather(bpr_v, [b3[2, sl]])
                dv = plsc.load_gather(den_v, [b3[0, sl]])
                eb[sl] = jnp.exp(bv + cv) / (dv + 1e-16)

            @pl.when(cid == 0)
            def _():
                pltpu.async_copy(tab0.at[b3.at[1]], rw, sm)

            @pl.when(cid == 1)
            def _():
                idx = b3.at[2] if use_rel_idx else b3.at[1]
                pltpu.async_copy(tab1.at[idx], rw, sm)

        def _finish(bufset):
            b3, eb, rw, sm = bufset

            @pl.when(cid == 0)
            def _():
                pltpu.make_async_copy(tab0.at[b3.at[1]], rw, sm).wait()

            @pl.when(cid == 1)
            def _():
                idx = b3.at[2] if use_rel_idx else b3.at[1]
                pltpu.make_async_copy(tab1.at[idx], rw, sm).wait()

            @plsc.parallel_loop(0, CHUNK, step=1, unroll=4)
            def _scale(e):
                al = plsc.load_gather(eb, [jnp.full((L,), e, jnp.int32)])
                for off in (0, 16, 32, 48, 64, 80, 96):
                    rw[e, pl.ds(off, L)] = rw[e, pl.ds(off, L)] * al

            pltpu.sync_copy(rw, acc.at[b3.at[0]], add=True)

        _issue(0, bA)

        def _pair(h, carry):
            g = 2 * h
            _issue(g + 1, bB)
            _finish(bA)
            _issue(g + 2, bA)
            _finish(bB)
            return carry

        lax.fori_loop(0, (NCHUNK - 1) // 2, _pair, 0)
        _finish(bA)
        plsc.subcore_barrier()

    def _dump_acc(out0, out1):
        sl = pl.ds(sid * ROWS_T, ROWS_T)

        @pl.when(cid == 0)
        def _():
            pltpu.sync_copy(acc.at[sl], out0.at[sl])

        @pl.when(cid == 1)
        def _():
            pltpu.sync_copy(acc.at[sl], out1.at[sl])

    _subpass(ha, hc, use_rel_idx=False)
    _dump_acc(outa0, outa1)
    _zero_acc()
    plsc.subcore_barrier()

    _subpass(hb, rtab, use_rel_idx=True)
    _dump_acc(outb0, outb1)


_sc_main = _make_sc_main(_sc_body)


def kernel(x, edge_index_all, rel_all, rel_emb, W1):
    i_arr = edge_index_all[0].astype(jnp.int32)
    j_arr = edge_index_all[1].astype(jnp.int32)
    r_arr = rel_all.astype(jnp.int32)
    ijr = jnp.stack([i_arr, j_arr, r_arr], axis=0)
    packed = (ijr.reshape(3, NS, NCHUNK, CHUNK)
              .transpose(1, 2, 0, 3)
              .reshape(NS * NCHUNK, 3, CHUNK))

    ha, hb, hc, rtab, cpr, bpr = _prep(x, rel_emb, W1)
    zrows = jnp.zeros((ROWS_T, 128), jnp.float32)
    zden = jnp.zeros((NPAD,), jnp.float32)

    outa0, outb0, outa1, outb1, dout = _sc_main(
        packed,
        cpr.reshape(-1), bpr.reshape(-1),
        ha, hb, hc, rtab, zrows, zden,
    )

    head = _head(ha, hb, hc, dout[:N_NODES].reshape(N_NODES, 1))
    return jnp.concatenate(
        [head, outb1[:N_NODES, :100], outa0[:N_NODES, :100],
         outb0[:N_NODES, :100], outa1[:N_NODES, :100]],
        axis=1,
    )


# speedup vs baseline: 7.4048x; 1.0048x over previous
import functools

import jax
import jax.numpy as jnp
from jax import lax
from jax.experimental import pallas as pl
from jax.experimental.pallas import tpu as pltpu
from jax.experimental.pallas import tpu_sc as plsc

N_NODES = 10000
N_EDGES = 160000
N_RELS = 1000
D_FEAT = 300
R_HID = 100

NC, NS, L = 2, 16, 16
EPT = N_EDGES // NS
CHUNK = 80
NCHUNK = EPT // CHUNK
NPAD = 10240
ROWS_T = NPAD // NS


def _prep_body(x_ref, rel_ref, w1_ref, ha_ref, hb_ref, hc_ref, rtab_ref,
               cpr_ref, bpr_ref):
    x = x_ref[...]
    h = jnp.where(x >= 0, x, 0.01 * x)
    zpad = jnp.zeros((h.shape[0], 28), jnp.float32)
    ha_ref[...] = jnp.concatenate([h[:, 0:100], zpad], axis=1)
    hb_ref[...] = jnp.concatenate([h[:, 100:200], zpad], axis=1)
    hc_ref[...] = jnp.concatenate([h[:, 200:300], zpad], axis=1)
    w1 = w1_ref[...]
    wc = w1[:, 400:700]
    cvec = jnp.dot(h, wc.T, preferred_element_type=jnp.float32)
    cpr_ref[...] = cvec - jnp.max(cvec)
    rel = rel_ref[...]
    r = jnp.where(rel >= 0, rel, 0.01 * rel)
    rtab_ref[...] = jnp.concatenate(
        [r, jnp.zeros((r.shape[0], 28), jnp.float32)], axis=1)
    wb = w1[:, 300:400]
    bvec = jnp.dot(r, wb.T, preferred_element_type=jnp.float32)
    bpr_ref[...] = bvec - jnp.max(bvec)


_prep = pl.pallas_call(
    _prep_body,
    out_shape=[
        jax.ShapeDtypeStruct((N_NODES, 128), jnp.float32),
        jax.ShapeDtypeStruct((N_NODES, 128), jnp.float32),
        jax.ShapeDtypeStruct((N_NODES, 128), jnp.float32),
        jax.ShapeDtypeStruct((N_RELS, 128), jnp.float32),
        jax.ShapeDtypeStruct((N_NODES, 1), jnp.float32),
        jax.ShapeDtypeStruct((N_RELS, 1), jnp.float32),
    ],
)


def _head_body(ha_ref, hb_ref, hc_ref, den_ref, out_ref):
    den = den_ref[...]
    s = den / (den + 1e-16)
    out_ref[:, 0:100] = ha_ref[:, 0:100] * s
    out_ref[:, 100:200] = hb_ref[:, 0:100] * s
    out_ref[:, 200:300] = hc_ref[:, 0:100] * s


_head = pl.pallas_call(
    _head_body,
    out_shape=jax.ShapeDtypeStruct((N_NODES, D_FEAT), jnp.float32),
)


_mesh = plsc.VectorSubcoreMesh(
    core_axis_name="c", subcore_axis_name="s", num_cores=NC, num_subcores=NS
)


def _make_sc_main(body, interpret=False):
    return pl.kernel(
        body,
        mesh=_mesh,
        interpret=interpret,
        compiler_params=pltpu.CompilerParams(
            needs_layout_passes=False, use_tc_tiling_on_sc=False
        ),
        out_type=[
        jax.ShapeDtypeStruct((NPAD, 128), jnp.float32),
        jax.ShapeDtypeStruct((NPAD, 128), jnp.float32),
        jax.ShapeDtypeStruct((NPAD, 128), jnp.float32),
        jax.ShapeDtypeStruct((NPAD, 128), jnp.float32),
        jax.ShapeDtypeStruct((NPAD,), jnp.float32),
    ],
    scratch_types=[
        pltpu.VMEM_SHARED((NPAD, 128), jnp.float32),
        pltpu.VMEM_SHARED((NPAD,), jnp.float32),
        pltpu.VMEM((N_NODES,), jnp.float32),
        pltpu.VMEM((N_RELS,), jnp.float32),
        pltpu.VMEM((NPAD,), jnp.float32),
        pltpu.VMEM((3, CHUNK), jnp.int32),
        pltpu.VMEM((CHUNK,), jnp.float32),
        pltpu.VMEM((CHUNK, 128), jnp.float32),
        pltpu.SemaphoreType.DMA,
        pltpu.VMEM((3, CHUNK), jnp.int32),
        pltpu.VMEM((CHUNK,), jnp.float32),
        pltpu.VMEM((CHUNK, 128), jnp.float32),
        pltpu.SemaphoreType.DMA,
        pltpu.SemaphoreType.DMA,
        pltpu.SemaphoreType.DMA,
        ],
    )


def _sc_body(packed, cpr, bpr, ha, hb, hc, rtab, zrows, zden,
             outa0, outb0, outa1, outb1, dout,
             acc, den_sp, cpr_v, bpr_v, den_v,
             buf3, ebuf, rows, sem,
             buf3B, ebufB, rowsB, semB, semS, semSB):
    cid = lax.axis_index("c")
    sid = lax.axis_index("s")

    bA = (buf3, ebuf, rows, sem, semS)
    bB = (buf3B, ebufB, rowsB, semB, semSB)

    def _zero_acc():
        pltpu.sync_copy(zrows, acc.at[pl.ds(sid * ROWS_T, ROWS_T)])

    def _load_idx(g, b3):
        pltpu.sync_copy(packed.at[sid * NCHUNK + g], b3)

    pltpu.sync_copy(cpr, cpr_v)
    pltpu.sync_copy(bpr, bpr_v)
    _zero_acc()
    pltpu.sync_copy(zden.at[pl.ds(sid * (NPAD // NS), NPAD // NS)],
                    den_sp.at[pl.ds(sid * (NPAD // NS), NPAD // NS)])
    plsc.subcore_barrier()

    def _p1_issue(g, bufset):
        b3, eb, _, _, _ = bufset
        _load_idx(g, b3)
        for k in range(CHUNK // L):
            sl = pl.ds(k * L, L)
            cv = plsc.load_gather(cpr_v, [b3[1, sl]])
            bv = plsc.load_gather(bpr_v, [b3[2, sl]])
            eb[sl] = jnp.exp(bv + cv)

    def _p1_start(bufset):
        b3, eb, _, sm, _ = bufset
        pltpu.async_copy(eb, den_sp.at[b3.at[0]], sm, add=True)

    def _p1_wait(bufset):
        b3, eb, _, sm, _ = bufset
        pltpu.make_async_copy(eb, den_sp.at[b3.at[0]], sm).wait()

    _p1_issue(0, bA)

    def _p1_pair(h, carry):
        g = 2 * h
        _p1_start(bA)
        _p1_issue(g + 1, bB)
        _p1_wait(bA)
        _p1_start(bB)
        _p1_issue(g + 2, bA)
        _p1_wait(bB)
        return carry

    lax.fori_loop(0, (NCHUNK - 1) // 2, _p1_pair, 0)
    _p1_start(bA)
    _p1_wait(bA)
    plsc.subcore_barrier()

    pltpu.sync_copy(den_sp, den_v)

    @pl.when(cid == 0)
    def _():
        pltpu.sync_copy(den_sp.at[pl.ds(sid * (NPAD // NS), NPAD // NS)],
                        dout.at[pl.ds(sid * (NPAD // NS), NPAD // NS)])

    def _subpass(tab0, tab1, use_rel_idx):
        def _issue(g, bufset):
            b3, eb, rw, sm, _ = bufset
            _load_idx(g, b3)
            for k in range(CHUNK // L):
                sl = pl.ds(k * L, L)
                cv = plsc.load_gather(cpr_v, [b3[1, sl]])
                bv = plsc.load_gather(bpr_v, [b3[2, sl]])
                dv = plsc.load_gather(den_v, [b3[0, sl]])
                eb[sl] = jnp.exp(bv + cv) / (dv + 1e-16)

            @pl.when(cid == 0)
            def _():
                pltpu.async_copy(tab0.at[b3.at[1]], rw, sm)

            @pl.when(cid == 1)
            def _():
                idx = b3.at[2] if use_rel_idx else b3.at[1]
                pltpu.async_copy(tab1.at[idx], rw, sm)

        def _finish(bufset):
            b3, eb, rw, sm, sms = bufset

            @pl.when(cid == 0)
            def _():
                pltpu.make_async_copy(tab0.at[b3.at[1]], rw, sm).wait()

            @pl.when(cid == 1)
            def _():
                idx = b3.at[2] if use_rel_idx else b3.at[1]
                pltpu.make_async_copy(tab1.at[idx], rw, sm).wait()

            @plsc.parallel_loop(0, CHUNK, step=1, unroll=4)
            def _scale(e):
                al = plsc.load_gather(eb, [jnp.full((L,), e, jnp.int32)])
                for off in (0, 16, 32, 48, 64, 80, 96):
                    rw[e, pl.ds(off, L)] = rw[e, pl.ds(off, L)] * al

            pltpu.async_copy(rw, acc.at[b3.at[0]], sms, add=True)

        def _wait_scat(bufset):
            b3, eb, rw, sm, sms = bufset
            pltpu.make_async_copy(rw, acc.at[b3.at[0]], sms).wait()

        _issue(0, bA)
        _issue(1, bB)

        def _pair(h, carry):
            g = 2 * h
            _finish(bA)
            _finish(bB)
            _wait_scat(bA)
            _issue(g + 2, bA)
            _wait_scat(bB)

            @pl.when(g + 3 < NCHUNK)
            def _():
                _issue(g + 3, bB)

            return carry

        lax.fori_loop(0, (NCHUNK - 1) // 2, _pair, 0)
        _finish(bA)
        _wait_scat(bA)
        plsc.subcore_barrier()

    def _dump_acc(out0, out1):
        sl = pl.ds(sid * ROWS_T, ROWS_T)

        @pl.when(cid == 0)
        def _():
            pltpu.sync_copy(acc.at[sl], out0.at[sl])

        @pl.when(cid == 1)
        def _():
            pltpu.sync_copy(acc.at[sl], out1.at[sl])

    _subpass(ha, hc, use_rel_idx=False)
    _dump_acc(outa0, outa1)
    _zero_acc()
    plsc.subcore_barrier()

    _subpass(hb, rtab, use_rel_idx=True)
    _dump_acc(outb0, outb1)


_sc_main = _make_sc_main(_sc_body)


def kernel(x, edge_index_all, rel_all, rel_emb, W1):
    i_arr = edge_index_all[0].astype(jnp.int32)
    j_arr = edge_index_all[1].astype(jnp.int32)
    r_arr = rel_all.astype(jnp.int32)
    ijr = jnp.stack([i_arr, j_arr, r_arr], axis=0)
    packed = (ijr.reshape(3, NS, NCHUNK, CHUNK)
              .transpose(1, 2, 0, 3)
              .reshape(NS * NCHUNK, 3, CHUNK))

    ha, hb, hc, rtab, cpr, bpr = _prep(x, rel_emb, W1)
    zrows = jnp.zeros((ROWS_T, 128), jnp.float32)
    zden = jnp.zeros((NPAD,), jnp.float32)

    outa0, outb0, outa1, outb1, dout = _sc_main(
        packed,
        cpr.reshape(-1), bpr.reshape(-1),
        ha, hb, hc, rtab, zrows, zden,
    )

    head = _head(ha, hb, hc, dout[:N_NODES].reshape(N_NODES, 1))
    return jnp.concatenate(
        [head, outb1[:N_NODES, :100], outa0[:N_NODES, :100],
         outb0[:N_NODES, :100], outa1[:N_NODES, :100]],
        axis=1,
    )
